# baseline (speedup 1.0000x reference)
import jax
import jax.numpy as jnp
import numpy as np
from jax import lax
from jax.experimental import pallas as pl
from jax.experimental.pallas import tpu as pltpu
from jax.experimental.pallas import tpu_sc as plsc

N = 10000
E = 320000
D_IN = 128
HID = 32
HEADS = 8
D_OUT = 128

N_PAD = 10240
NE = E + N
NT = 32
EPT = 82 * 128
E_PAD = NT * EPT
NA = 10016
ROWS_A = NA // 16
BR = 512

C1 = 64
CD = 128
C2 = 128

f32 = jnp.float32
i32 = jnp.int32

_cols128 = np.arange(128)[None, :] // HID
_rows16 = np.arange(16)[:, None]
E1A = (_cols128 == _rows16).astype(np.float32)
E1B = (_cols128 + 4 == _rows16).astype(np.float32)
E2 = (np.arange(32)[None, :] * 0 == _rows16).astype(np.float32)


def _embed_attn(a):
    h_, c_ = a.shape
    rows = jnp.arange(h_ * c_)
    return jnp.zeros((h_ * c_, 16), f32).at[rows, rows // c_].set(
        a.reshape(-1).astype(f32))


def _pre1_body(x_ref, w1_ref, as_w, ad_w, h12_ref, as_ref, ad_ref):
    h = jnp.dot(x_ref[...], w1_ref[...], preferred_element_type=f32)
    h12_ref[0] = h[:, :128]
    h12_ref[1] = h[:, 128:]
    as_ref[...] = jnp.dot(h, as_w[...], preferred_element_type=f32)
    ad_ref[...] = jnp.dot(h, ad_w[...], preferred_element_type=f32)


def _pre1(x_p, W1, As, Ad):
    full = lambda i: (0, 0)
    row = lambda i: (i, 0)
    return pl.pallas_call(
        _pre1_body,
        grid=(N_PAD // BR,),
        in_specs=[pl.BlockSpec((BR, D_IN), row),
                  pl.BlockSpec((D_IN, 2 * 128), full),
                  pl.BlockSpec((2 * 128, 16), full),
                  pl.BlockSpec((2 * 128, 16), full)],
        out_specs=[pl.BlockSpec((2, BR, 128), lambda i: (0, i, 0)),
                   pl.BlockSpec((BR, 16), row),
                   pl.BlockSpec((BR, 16), row)],
        out_shape=[jax.ShapeDtypeStruct((2, N_PAD, 128), f32),
                   jax.ShapeDtypeStruct((N_PAD, 16), f32),
                   jax.ShapeDtypeStruct((N_PAD, 16), f32)],
    )(x_p, W1, As, Ad)


def _elu(u):
    return jnp.where(u > 0, u, jnp.exp(jnp.minimum(u, 0.0)) - 1.0)


def _mid_body(pa_ref, pb_ref, den_ref, w2a_ref, w2b_ref, b1a_ref, b1b_ref,
              e1a_ref, e1b_ref, as_w, ad_w, h2_ref, as_ref, ad_ref):
    den = den_ref[0] + den_ref[1]
    da = jnp.maximum(jnp.dot(den, e1a_ref[...], preferred_element_type=f32),
                     1e-30)
    db = jnp.maximum(jnp.dot(den, e1b_ref[...], preferred_element_type=f32),
                     1e-30)
    ua = (pa_ref[0] + pa_ref[1]) / da + b1a_ref[...]
    ub = (pb_ref[0] + pb_ref[1]) / db + b1b_ref[...]
    ha = _elu(ua)
    hb = _elu(ub)
    h2 = (jnp.dot(ha, w2a_ref[...], preferred_element_type=f32)
          + jnp.dot(hb, w2b_ref[...], preferred_element_type=f32))
    h2_ref[...] = h2
    as_ref[...] = jnp.dot(h2, as_w[...], preferred_element_type=f32)
    ad_ref[...] = jnp.dot(h2, ad_w[...], preferred_element_type=f32)


def _mid(pa, pb, den, W2a, W2b, b1a, b1b, As2, Ad2):
    full = lambda i: (0, 0)
    row3 = lambda i: (0, i, 0)
    row = lambda i: (i, 0)
    return pl.pallas_call(
        _mid_body,
        grid=(N_PAD // BR,),
        in_specs=[pl.BlockSpec((2, BR, 128), row3),
                  pl.BlockSpec((2, BR, 128), row3),
                  pl.BlockSpec((2, BR, 16), row3),
                  pl.BlockSpec((128, HID), full),
                  pl.BlockSpec((128, HID), full),
                  pl.BlockSpec((1, 128), full),
                  pl.BlockSpec((1, 128), full),
                  pl.BlockSpec((16, 128), full),
                  pl.BlockSpec((16, 128), full),
                  pl.BlockSpec((HID, 16), full),
                  pl.BlockSpec((HID, 16), full)],
        out_specs=[pl.BlockSpec((BR, HID), row),
                   pl.BlockSpec((BR, 16), row),
                   pl.BlockSpec((BR, 16), row)],
        out_shape=[jax.ShapeDtypeStruct((N_PAD, HID), f32),
                   jax.ShapeDtypeStruct((N_PAD, 16), f32),
                   jax.ShapeDtypeStruct((N_PAD, 16), f32)],
    )(pa, pb, den, W2a, W2b, b1a, b1b, E1A, E1B, As2, Ad2)


def _post_body(p_ref, den_ref, e2_ref, b2_ref, wout_ref, bout_ref, y_ref):
    den = den_ref[0] + den_ref[1]
    d = jnp.maximum(jnp.dot(den, e2_ref[...], preferred_element_type=f32),
                    1e-30)
    u = (p_ref[0] + p_ref[1]) / d + b2_ref[...]
    h = _elu(u)
    y_ref[...] = (jnp.dot(h, wout_ref[...], preferred_element_type=f32)
                  + bout_ref[...])


def _post(p2, den2, b2r, Wout, boutr):
    full = lambda i: (0, 0)
    row3 = lambda i: (0, i, 0)
    row = lambda i: (i, 0)
    return pl.pallas_call(
        _post_body,
        grid=(N_PAD // BR,),
        in_specs=[pl.BlockSpec((2, BR, HID), row3),
                  pl.BlockSpec((2, BR, 16), row3),
                  pl.BlockSpec((16, HID), full),
                  pl.BlockSpec((1, HID), full),
                  pl.BlockSpec((HID, D_OUT), full),
                  pl.BlockSpec((1, D_OUT), full)],
        out_specs=[pl.BlockSpec((BR, D_OUT), row)],
        out_shape=[jax.ShapeDtypeStruct((N_PAD, D_OUT), f32)],
    )(p2, den2, E2, b2r, Wout, boutr)


def _make_edge(d_feat, head_base, compute_den, with_msg, ck, phases=1):
    nvec = d_feat // 16
    ncht = EPT // ck
    nchp = ncht // phases
    mesh = plsc.VectorSubcoreMesh(core_axis_name="c", subcore_axis_name="s")

    out_type = []
    if with_msg:
        out_type.append(jax.ShapeDtypeStruct((2, N_PAD, d_feat), f32))
    if compute_den:
        out_type.append(jax.ShapeDtypeStruct((2, N_PAD, 16), f32))

    bufw = 2 + (1 if compute_den else 0) + (2 if with_msg else 0) + 2
    scratch = [
        pltpu.VMEM((nchp, ck), i32),
        pltpu.VMEM((nchp, ck), i32),
    ]
    for _b in range(2):
        scratch += [pltpu.VMEM((ck, 16), f32),
                    pltpu.VMEM((ck, 16), f32)]
        if compute_den:
            scratch.append(pltpu.VMEM((ck, 16), f32))
        if with_msg:
            scratch += [pltpu.VMEM((ck, d_feat), f32),
                        pltpu.VMEM((ck, d_feat), f32)]
        scratch += [pltpu.SemaphoreType.DMA, pltpu.SemaphoreType.DMA]
    if with_msg:
        scratch.append(pltpu.VMEM_SHARED((NA, d_feat), f32))
    if compute_den:
        scratch.append(pltpu.VMEM_SHARED((NA, 16), f32))

    def body(*args):
        a = list(args)
        h_hbm = a.pop(0) if with_msg else None
        sa_hbm, da_hbm, src_hbm, dst_hbm = a[:4]
        a = a[4:]
        out_hbm = a.pop(0) if with_msg else None
        den_hbm = a.pop(0) if compute_den else None
        idxs, idxd = a[0], a[1]
        bufs = [a[2 + bufw * b:2 + bufw * (b + 1)] for b in range(2)]
        a = a[2 + 2 * bufw:]
        acc = a.pop(0) if with_msg else None
        denacc = a.pop(0) if compute_den else None

        cid = lax.axis_index("c")
        sid = lax.axis_index("s")
        wid = sid * 2 + cid

        def parts(buf):
            sa_v, da_v = buf[0], buf[1]
            k = 2
            w_v = buf[k] if compute_den else None
            k += 1 if compute_den else 0
            h_v = buf[k] if with_msg else None
            msg_v = buf[k + 1] if with_msg else None
            return sa_v, da_v, w_v, h_v, msg_v, buf[-2], buf[-1]

        _, _, w0, _, msg0, _, _ = parts(bufs[0])

        @pl.loop(0, ck)
        def _(r):
            if with_msg:
                for j in range(nvec):
                    msg0[r, pl.ds(j * 16, 16)] = jnp.zeros((16,), f32)
            if compute_den:
                w0[r, :] = jnp.zeros((16,), f32)

        rb = sid * ROWS_A
        nfull, nrem = ROWS_A // ck, ROWS_A % ck
        for k in range(nfull):
            if with_msg:
                pltpu.sync_copy(msg0, acc.at[pl.ds(rb + k * ck, ck)])
            if compute_den:
                pltpu.sync_copy(w0, denacc.at[pl.ds(rb + k * ck, ck)])
        if nrem:
            if with_msg:
                pltpu.sync_copy(msg0.at[pl.ds(0, nrem)],
                                acc.at[pl.ds(rb + nfull * ck, nrem)])
            if compute_den:
                pltpu.sync_copy(w0.at[pl.ds(0, nrem)],
                                denacc.at[pl.ds(rb + nfull * ck, nrem)])
        plsc.subcore_barrier()

        def start_gathers(c, buf):
            sa_v, da_v, _, h_v, _, gsem, _ = parts(buf)
            pltpu.async_copy(sa_hbm.at[idxs.at[c]], sa_v, gsem)
            pltpu.async_copy(da_hbm.at[idxd.at[c]], da_v, gsem)
            if with_msg:
                pltpu.async_copy(h_hbm.at[idxs.at[c]], h_v, gsem)

        def wait_gathers(c, buf):
            sa_v, da_v, _, h_v, _, gsem, _ = parts(buf)
            pltpu.make_async_copy(sa_hbm.at[idxs.at[c]], sa_v, gsem).wait()
            pltpu.make_async_copy(da_hbm.at[idxd.at[c]], da_v, gsem).wait()
            if with_msg:
                pltpu.make_async_copy(h_hbm.at[idxs.at[c]], h_v,
                                      gsem).wait()

        def compute(c, buf):
            sa_v, da_v, w_v, h_v, msg_v, _, _ = parts(buf)

            @pl.loop(0, ck)
            def _(r):
                e = sa_v[r, :] + da_v[r, :]
                e = jnp.maximum(e, 0.2 * e)
                w = jnp.exp(e)
                if compute_den:
                    w_v[r, :] = w
                if with_msg:
                    for j in range(nvec):
                        lane = head_base + j // 2
                        wj = w.at[jnp.full((16,), lane, i32)].get(
                            mode="promise_in_bounds")
                        msg_v[r, pl.ds(j * 16, 16)] = (
                            h_v[r, pl.ds(j * 16, 16)] * wj)

        def start_scatters(c, buf):
            _, _, w_v, _, msg_v, _, ssem = parts(buf)
            if with_msg:
                pltpu.async_copy(msg_v, acc.at[idxd.at[c]], ssem, add=True)
            if compute_den:
                pltpu.async_copy(w_v, denacc.at[idxd.at[c]], ssem,
                                 add=True)

        def wait_scatters(c, buf):
            _, _, w_v, _, msg_v, _, ssem = parts(buf)
            if with_msg:
                pltpu.make_async_copy(msg_v, acc.at[idxd.at[c]],
                                      ssem).wait()
            if compute_den:
                pltpu.make_async_copy(w_v, denacc.at[idxd.at[c]],
                                      ssem).wait()

        for p in range(phases):
            pltpu.sync_copy(src_hbm.at[wid, pl.ds(p * nchp, nchp)], idxs)
            pltpu.sync_copy(dst_hbm.at[wid, pl.ds(p * nchp, nchp)], idxd)
            start_gathers(0, bufs[0])

            @pl.loop(0, nchp, step=2)
            def _(c):
                start_gathers(c + 1, bufs[1])

                @pl.when(c >= 2)
                def _():
                    wait_scatters(c - 2, bufs[0])
                wait_gathers(c, bufs[0])
                compute(c, bufs[0])
                start_scatters(c, bufs[0])

                @pl.when(c + 2 < nchp)
                def _():
                    start_gathers(c + 2, bufs[0])

                @pl.when(c >= 2)
                def _():
                    wait_scatters(c - 1, bufs[1])
                wait_gathers(c + 1, bufs[1])
                compute(c + 1, bufs[1])
                start_scatters(c + 1, bufs[1])

            wait_scatters(nchp - 2, bufs[0])
            wait_scatters(nchp - 1, bufs[1])

        plsc.subcore_barrier()
        if with_msg:
            pltpu.sync_copy(acc.at[pl.ds(rb, ROWS_A)],
                            out_hbm.at[cid, pl.ds(rb, ROWS_A)])
        if compute_den:
            pltpu.sync_copy(denacc.at[pl.ds(rb, ROWS_A)],
                            den_hbm.at[cid, pl.ds(rb, ROWS_A)])

    return pl.kernel(body, out_type=tuple(out_type), mesh=mesh,
                     scratch_types=tuple(scratch),
                     compiler_params=pltpu.CompilerParams(
                         use_tc_tiling_on_sc=False))


_den1 = _make_edge(16, 0, True, False, CD)
_edge1a = _make_edge(128, 0, False, True, C1, 2)
_edge1b = _make_edge(128, 4, False, True, C1, 2)
_edge2 = _make_edge(HID, 0, True, True, C2)


def kernel(x, edge_index, W1, a_src1, a_dst1, b1, W2, a_src2, a_dst2, b2,
           Wout, bout):
    loop = jnp.arange(N, dtype=i32)
    src = jnp.concatenate([edge_index[0].astype(i32), loop])
    dst = jnp.concatenate([edge_index[1].astype(i32), loop])
    npad = E_PAD - NE
    src_f = jnp.concatenate([src, jnp.zeros((npad,), i32)])
    dst_f = jnp.concatenate([dst, jnp.full((npad,), N, i32)])

    def idx_pair(ck):
        return (src_f.reshape(NT, EPT // ck, ck),
                dst_f.reshape(NT, EPT // ck, ck))

    src_1, dst_1 = idx_pair(C1)
    src_d, dst_d = idx_pair(CD)
    src_2, dst_2 = idx_pair(C2)

    x_p = jnp.pad(x.astype(f32), ((0, N_PAD - N), (0, 0)))
    As1 = _embed_attn(a_src1)
    Ad1 = _embed_attn(a_dst1)
    As2 = _embed_attn(a_src2)
    Ad2 = _embed_attn(a_dst2)

    h12, asrc1, adst1 = _pre1(x_p, W1.astype(f32), As1, Ad1)

    (den1,) = _den1(asrc1, adst1, src_d, dst_d)
    (pa,) = _edge1a(h12[0], asrc1, adst1, src_1, dst_1)
    (pb,) = _edge1b(h12[1], asrc1, adst1, src_1, dst_1)

    h2, asrc2, adst2 = _mid(pa, pb, den1,
                            W2[:128].astype(f32), W2[128:].astype(f32),
                            b1[:128].reshape(1, 128).astype(f32),
                            b1[128:].reshape(1, 128).astype(f32),
                            As2, Ad2)

    p2, den2 = _edge2(h2, asrc2, adst2, src_2, dst_2)

    (y,) = _post(p2, den2, b2.reshape(1, HID).astype(f32),
                 Wout.astype(f32), bout.reshape(1, D_OUT).astype(f32))
    return y[:N]


# speedup vs baseline: 1.2991x; 1.2991x over previous
import jax
import jax.numpy as jnp
import numpy as np
from jax import lax
from jax.experimental import pallas as pl
from jax.experimental.pallas import tpu as pltpu
from jax.experimental.pallas import tpu_sc as plsc

N = 10000
E = 320000
D_IN = 128
HID = 32
HEADS = 8
D_OUT = 128

N_PAD = 10240
NE = E + N
NT = 32
EPT = 82 * 128
E_PAD = NT * EPT
NA = 10016
ROWS_A = NA // 16
BR = 512

C1 = 32
C2 = 128

f32 = jnp.float32
i32 = jnp.int32

_cols128 = np.arange(128)[None, :] // HID
_rows16 = np.arange(16)[:, None]
E1A = (_cols128 == _rows16).astype(np.float32)
E1B = (_cols128 + 4 == _rows16).astype(np.float32)
E2 = (np.arange(32)[None, :] * 0 == _rows16).astype(np.float32)

_pb32 = np.concatenate([np.arange(0, 32, 2), np.arange(1, 32, 2)])
PERM128 = np.concatenate([b * 32 + _pb32 for b in range(4)])


def _embed_attn(a):
    h_, c_ = a.shape
    rows = jnp.arange(h_ * c_)
    return jnp.zeros((h_ * c_, 16), f32).at[rows, rows // c_].set(
        a.reshape(-1).astype(f32))


def _pre1_body(x_ref, w1_ref, as_w, ad_w, h12_ref, as_ref, ad_ref):
    h = jnp.dot(x_ref[...], w1_ref[...], preferred_element_type=f32)
    h12_ref[0] = h[:, :128].astype(jnp.bfloat16)
    h12_ref[1] = h[:, 128:].astype(jnp.bfloat16)
    as_ref[...] = jnp.dot(h, as_w[...], preferred_element_type=f32)
    ad_ref[...] = jnp.dot(h, ad_w[...], preferred_element_type=f32)


def _pre1(x_p, W1, As, Ad):
    full = lambda i: (0, 0)
    row = lambda i: (i, 0)
    return pl.pallas_call(
        _pre1_body,
        grid=(N_PAD // BR,),
        in_specs=[pl.BlockSpec((BR, D_IN), row),
                  pl.BlockSpec((D_IN, 2 * 128), full),
                  pl.BlockSpec((2 * 128, 16), full),
                  pl.BlockSpec((2 * 128, 16), full)],
        out_specs=[pl.BlockSpec((2, BR, 128), lambda i: (0, i, 0)),
                   pl.BlockSpec((BR, 16), row),
                   pl.BlockSpec((BR, 16), row)],
        out_shape=[jax.ShapeDtypeStruct((2, N_PAD, 128), jnp.bfloat16),
                   jax.ShapeDtypeStruct((N_PAD, 16), f32),
                   jax.ShapeDtypeStruct((N_PAD, 16), f32)],
    )(x_p, W1, As, Ad)


def _elu(u):
    return jnp.where(u > 0, u, jnp.exp(jnp.minimum(u, 0.0)) - 1.0)


def _mid_body(pa_ref, pb_ref, den_ref, w2a_ref, w2b_ref, b1a_ref, b1b_ref,
              e1a_ref, e1b_ref, as_w, ad_w, h2_ref, as_ref, ad_ref):
    den = den_ref[0] + den_ref[1]
    da = jnp.maximum(jnp.dot(den, e1a_ref[...], preferred_element_type=f32),
                     1e-30)
    db = jnp.maximum(jnp.dot(den, e1b_ref[...], preferred_element_type=f32),
                     1e-30)
    ua = (pa_ref[0] + pa_ref[1]) / da + b1a_ref[...]
    ub = (pb_ref[0] + pb_ref[1]) / db + b1b_ref[...]
    ha = _elu(ua)
    hb = _elu(ub)
    h2 = (jnp.dot(ha, w2a_ref[...], preferred_element_type=f32)
          + jnp.dot(hb, w2b_ref[...], preferred_element_type=f32))
    h2_ref[...] = h2
    as_ref[...] = jnp.dot(h2, as_w[...], preferred_element_type=f32)
    ad_ref[...] = jnp.dot(h2, ad_w[...], preferred_element_type=f32)


def _mid(pa, pb, den, W2a, W2b, b1a, b1b, As2, Ad2):
    full = lambda i: (0, 0)
    row3 = lambda i: (0, i, 0)
    row = lambda i: (i, 0)
    return pl.pallas_call(
        _mid_body,
        grid=(N_PAD // BR,),
        in_specs=[pl.BlockSpec((2, BR, 128), row3),
                  pl.BlockSpec((2, BR, 128), row3),
                  pl.BlockSpec((2, BR, 16), row3),
                  pl.BlockSpec((128, HID), full),
                  pl.BlockSpec((128, HID), full),
                  pl.BlockSpec((1, 128), full),
                  pl.BlockSpec((1, 128), full),
                  pl.BlockSpec((16, 128), full),
                  pl.BlockSpec((16, 128), full),
                  pl.BlockSpec((HID, 16), full),
                  pl.BlockSpec((HID, 16), full)],
        out_specs=[pl.BlockSpec((BR, HID), row),
                   pl.BlockSpec((BR, 16), row),
                   pl.BlockSpec((BR, 16), row)],
        out_shape=[jax.ShapeDtypeStruct((N_PAD, HID), f32),
                   jax.ShapeDtypeStruct((N_PAD, 16), f32),
                   jax.ShapeDtypeStruct((N_PAD, 16), f32)],
    )(pa, pb, den, W2a, W2b, b1a, b1b, E1A, E1B, As2, Ad2)


def _post_body(p_ref, den_ref, e2_ref, b2_ref, wout_ref, bout_ref, y_ref):
    den = den_ref[0] + den_ref[1]
    d = jnp.maximum(jnp.dot(den, e2_ref[...], preferred_element_type=f32),
                    1e-30)
    u = (p_ref[0] + p_ref[1]) / d + b2_ref[...]
    h = _elu(u)
    y_ref[...] = (jnp.dot(h, wout_ref[...], preferred_element_type=f32)
                  + bout_ref[...])


def _post(p2, den2, b2r, Wout, boutr):
    full = lambda i: (0, 0)
    row3 = lambda i: (0, i, 0)
    row = lambda i: (i, 0)
    return pl.pallas_call(
        _post_body,
        grid=(N_PAD // BR,),
        in_specs=[pl.BlockSpec((2, BR, HID), row3),
                  pl.BlockSpec((2, BR, 16), row3),
                  pl.BlockSpec((16, HID), full),
                  pl.BlockSpec((1, HID), full),
                  pl.BlockSpec((HID, D_OUT), full),
                  pl.BlockSpec((1, D_OUT), full)],
        out_specs=[pl.BlockSpec((BR, D_OUT), row)],
        out_shape=[jax.ShapeDtypeStruct((N_PAD, D_OUT), f32)],
    )(p2, den2, E2, b2r, Wout, boutr)


def _make_edge(d_feat, head_base, compute_den, with_msg, ck, phases=1,
               h_bf16=False):
    nvec = d_feat // 16
    ncht = EPT // ck
    nchp = ncht // phases
    mesh = plsc.VectorSubcoreMesh(core_axis_name="c", subcore_axis_name="s")

    out_type = []
    if with_msg:
        out_type.append(jax.ShapeDtypeStruct((2, N_PAD, d_feat), f32))
    if compute_den:
        out_type.append(jax.ShapeDtypeStruct((2, N_PAD, 16), f32))

    bufw = 2 + (1 if compute_den else 0) + (2 if with_msg else 0) + 2
    scratch = [
        pltpu.VMEM((nchp, ck), i32),
        pltpu.VMEM((nchp, ck), i32),
    ]
    for _b in range(2):
        scratch += [pltpu.VMEM((ck, 16), f32),
                    pltpu.VMEM((ck, 16), f32)]
        if compute_den:
            scratch.append(pltpu.VMEM((ck, 16), f32))
        if with_msg:
            hdt = jnp.bfloat16 if h_bf16 else f32
            scratch += [pltpu.VMEM((ck, d_feat), hdt),
                        pltpu.VMEM((ck, d_feat), f32)]
        scratch += [pltpu.SemaphoreType.DMA, pltpu.SemaphoreType.DMA]
    if with_msg:
        scratch.append(pltpu.VMEM_SHARED((NA, d_feat), f32))
    if compute_den:
        scratch.append(pltpu.VMEM_SHARED((NA, 16), f32))

    def body(*args):
        a = list(args)
        h_hbm = a.pop(0) if with_msg else None
        sa_hbm, da_hbm, src_hbm, dst_hbm = a[:4]
        a = a[4:]
        out_hbm = a.pop(0) if with_msg else None
        den_hbm = a.pop(0) if compute_den else None
        idxs, idxd = a[0], a[1]
        bufs = [a[2 + bufw * b:2 + bufw * (b + 1)] for b in range(2)]
        a = a[2 + 2 * bufw:]
        acc = a.pop(0) if with_msg else None
        denacc = a.pop(0) if compute_den else None

        cid = lax.axis_index("c")
        sid = lax.axis_index("s")
        wid = sid * 2 + cid

        def parts(buf):
            sa_v, da_v = buf[0], buf[1]
            k = 2
            w_v = buf[k] if compute_den else None
            k += 1 if compute_den else 0
            h_v = buf[k] if with_msg else None
            msg_v = buf[k + 1] if with_msg else None
            return sa_v, da_v, w_v, h_v, msg_v, buf[-2], buf[-1]

        _, _, w0, _, msg0, _, _ = parts(bufs[0])

        @pl.loop(0, ck)
        def _(r):
            if with_msg:
                for j in range(nvec):
                    msg0[r, pl.ds(j * 16, 16)] = jnp.zeros((16,), f32)
            if compute_den:
                w0[r, :] = jnp.zeros((16,), f32)

        rb = sid * ROWS_A
        nfull, nrem = ROWS_A // ck, ROWS_A % ck
        for k in range(nfull):
            if with_msg:
                pltpu.sync_copy(msg0, acc.at[pl.ds(rb + k * ck, ck)])
            if compute_den:
                pltpu.sync_copy(w0, denacc.at[pl.ds(rb + k * ck, ck)])
        if nrem:
            if with_msg:
                pltpu.sync_copy(msg0.at[pl.ds(0, nrem)],
                                acc.at[pl.ds(rb + nfull * ck, nrem)])
            if compute_den:
                pltpu.sync_copy(w0.at[pl.ds(0, nrem)],
                                denacc.at[pl.ds(rb + nfull * ck, nrem)])
        plsc.subcore_barrier()

        def start_gathers(c, buf):
            sa_v, da_v, _, h_v, _, gsem, _ = parts(buf)
            pltpu.async_copy(sa_hbm.at[idxs.at[c]], sa_v, gsem)
            pltpu.async_copy(da_hbm.at[idxd.at[c]], da_v, gsem)
            if with_msg:
                pltpu.async_copy(h_hbm.at[idxs.at[c]], h_v, gsem)

        def wait_gathers(c, buf):
            sa_v, da_v, _, h_v, _, gsem, _ = parts(buf)
            pltpu.make_async_copy(sa_hbm.at[idxs.at[c]], sa_v, gsem).wait()
            pltpu.make_async_copy(da_hbm.at[idxd.at[c]], da_v, gsem).wait()
            if with_msg:
                pltpu.make_async_copy(h_hbm.at[idxs.at[c]], h_v,
                                      gsem).wait()

        def compute(c, buf):
            sa_v, da_v, w_v, h_v, msg_v, _, _ = parts(buf)

            @pl.loop(0, ck)
            def _(r):
                e = sa_v[r, :] + da_v[r, :]
                e = jnp.maximum(e, 0.2 * e)
                w = jnp.exp(e)
                if compute_den:
                    w_v[r, :] = w
                if with_msg and h_bf16:
                    for j in range(nvec // 2):
                        lane = head_base + j
                        wj = w.at[jnp.full((16,), lane, i32)].get(
                            mode="promise_in_bounds")
                        pair = plsc.bitcast(h_v[r, pl.ds(j * 32, 32)], i32)
                        lo = plsc.bitcast(pair << 16, f32)
                        hi = plsc.bitcast(
                            pair & jnp.int32(-65536), f32)
                        msg_v[r, pl.ds(j * 32, 16)] = lo * wj
                        msg_v[r, pl.ds(j * 32 + 16, 16)] = hi * wj
                elif with_msg:
                    for j in range(nvec):
                        lane = head_base + j // 2
                        wj = w.at[jnp.full((16,), lane, i32)].get(
                            mode="promise_in_bounds")
                        msg_v[r, pl.ds(j * 16, 16)] = (
                            h_v[r, pl.ds(j * 16, 16)] * wj)

        def start_scatters(c, buf):
            _, _, w_v, _, msg_v, _, ssem = parts(buf)
            if with_msg:
                pltpu.async_copy(msg_v, acc.at[idxd.at[c]], ssem, add=True)
            if compute_den:
                pltpu.async_copy(w_v, denacc.at[idxd.at[c]], ssem,
                                 add=True)

        def wait_scatters(c, buf):
            _, _, w_v, _, msg_v, _, ssem = parts(buf)
            if with_msg:
                pltpu.make_async_copy(msg_v, acc.at[idxd.at[c]],
                                      ssem).wait()
            if compute_den:
                pltpu.make_async_copy(w_v, denacc.at[idxd.at[c]],
                                      ssem).wait()

        for p in range(phases):
            pltpu.sync_copy(src_hbm.at[wid, pl.ds(p * nchp, nchp)], idxs)
            pltpu.sync_copy(dst_hbm.at[wid, pl.ds(p * nchp, nchp)], idxd)
            start_gathers(0, bufs[0])

            @pl.loop(0, nchp, step=2)
            def _(c):
                start_gathers(c + 1, bufs[1])

                @pl.when(c >= 2)
                def _():
                    wait_scatters(c - 2, bufs[0])
                wait_gathers(c, bufs[0])
                compute(c, bufs[0])
                start_scatters(c, bufs[0])

                @pl.when(c + 2 < nchp)
                def _():
                    start_gathers(c + 2, bufs[0])

                @pl.when(c >= 2)
                def _():
                    wait_scatters(c - 1, bufs[1])
                wait_gathers(c + 1, bufs[1])
                compute(c + 1, bufs[1])
                start_scatters(c + 1, bufs[1])

            wait_scatters(nchp - 2, bufs[0])
            wait_scatters(nchp - 1, bufs[1])

        plsc.subcore_barrier()
        if with_msg:
            pltpu.sync_copy(acc.at[pl.ds(rb, ROWS_A)],
                            out_hbm.at[cid, pl.ds(rb, ROWS_A)])
        if compute_den:
            pltpu.sync_copy(denacc.at[pl.ds(rb, ROWS_A)],
                            den_hbm.at[cid, pl.ds(rb, ROWS_A)])

    import dataclasses
    cp = pltpu.CompilerParams(use_tc_tiling_on_sc=False)
    if h_bf16 and "needs_layout_passes" in pltpu.CompilerParams.__dataclass_fields__:
        cp = dataclasses.replace(cp, needs_layout_passes=False)
    return pl.kernel(body, out_type=tuple(out_type), mesh=mesh,
                     scratch_types=tuple(scratch),
                     compiler_params=cp)


_edge1a = _make_edge(128, 0, True, True, C1,
                     h_bf16=True)
_edge1b = _make_edge(128, 4, False, True, C1,
                     h_bf16=True)
_edge2 = _make_edge(HID, 0, True, True, C2)


def kernel(x, edge_index, W1, a_src1, a_dst1, b1, W2, a_src2, a_dst2, b2,
           Wout, bout):
    loop = jnp.arange(N, dtype=i32)
    src = jnp.concatenate([edge_index[0].astype(i32), loop])
    dst = jnp.concatenate([edge_index[1].astype(i32), loop])
    npad = E_PAD - NE
    src_f = jnp.concatenate([src, jnp.zeros((npad,), i32)])
    dst_f = jnp.concatenate([dst, jnp.full((npad,), N, i32)])

    def idx_pair(ck):
        return (src_f.reshape(NT, EPT // ck, ck),
                dst_f.reshape(NT, EPT // ck, ck))

    src_1, dst_1 = idx_pair(C1)
    src_2, dst_2 = idx_pair(C2)

    x_p = jnp.pad(x.astype(f32), ((0, N_PAD - N), (0, 0)))
    As1 = _embed_attn(a_src1)
    Ad1 = _embed_attn(a_dst1)
    As2 = _embed_attn(a_src2)
    Ad2 = _embed_attn(a_dst2)

    h12, asrc1, adst1 = _pre1(x_p, W1.astype(f32), As1, Ad1)

    pa, den1 = _edge1a(h12[0], asrc1, adst1, src_1, dst_1)
    (pb,) = _edge1b(h12[1], asrc1, adst1, src_1, dst_1)

    h2, asrc2, adst2 = _mid(pa, pb, den1,
                            W2[:128][PERM128].astype(f32),
                            W2[128:][PERM128].astype(f32),
                            b1[:128][PERM128].reshape(1, 128).astype(f32),
                            b1[128:][PERM128].reshape(1, 128).astype(f32),
                            As2, Ad2)

    p2, den2 = _edge2(h2, asrc2, adst2, src_2, dst_2)

    (y,) = _post(p2, den2, b2.reshape(1, HID).astype(f32),
                 Wout.astype(f32), bout.reshape(1, D_OUT).astype(f32))
    return y[:N]


# speedup vs baseline: 1.3743x; 1.0579x over previous
import jax
import jax.numpy as jnp
import numpy as np
from jax import lax
from jax.experimental import pallas as pl
from jax.experimental.pallas import tpu as pltpu
from jax.experimental.pallas import tpu_sc as plsc

N = 10000
E = 320000
D_IN = 128
HID = 32
HEADS = 8
D_OUT = 128

N_PAD = 10240
NE = E + N
NT = 32
EPT = 82 * 128
E_PAD = NT * EPT
NA = 10016
ROWS_A = NA // 16
BR = 512

C1 = 32
C2 = 128

f32 = jnp.float32
i32 = jnp.int32

_cols128 = np.arange(128)[None, :] // HID
_rows16 = np.arange(16)[:, None]
E1A = (_cols128 == _rows16).astype(np.float32)
E1B = (_cols128 + 4 == _rows16).astype(np.float32)
E2 = (np.arange(32)[None, :] * 0 == _rows16).astype(np.float32)

_pb32 = np.concatenate([np.arange(0, 32, 2), np.arange(1, 32, 2)])
PERM128 = np.concatenate([b * 32 + _pb32 for b in range(4)])


def _embed_attn(a):
    h_, c_ = a.shape
    rows = jnp.arange(h_ * c_)
    return jnp.zeros((h_ * c_, 16), f32).at[rows, rows // c_].set(
        a.reshape(-1).astype(f32))


def _pre1_body(x_ref, w1_ref, as_w, ad_w, h12_ref, as_ref, ad_ref):
    h = jnp.dot(x_ref[...], w1_ref[...], preferred_element_type=f32)
    h12_ref[0] = h[:, :128].astype(jnp.bfloat16)
    h12_ref[1] = h[:, 128:].astype(jnp.bfloat16)
    as_ref[...] = jnp.dot(h, as_w[...], preferred_element_type=f32)
    ad_ref[...] = jnp.dot(h, ad_w[...], preferred_element_type=f32)


def _pre1(x_p, W1, As, Ad):
    full = lambda i: (0, 0)
    row = lambda i: (i, 0)
    return pl.pallas_call(
        _pre1_body,
        grid=(N_PAD // BR,),
        in_specs=[pl.BlockSpec((BR, D_IN), row),
                  pl.BlockSpec((D_IN, 2 * 128), full),
                  pl.BlockSpec((2 * 128, 16), full),
                  pl.BlockSpec((2 * 128, 16), full)],
        out_specs=[pl.BlockSpec((2, BR, 128), lambda i: (0, i, 0)),
                   pl.BlockSpec((BR, 16), row),
                   pl.BlockSpec((BR, 16), row)],
        out_shape=[jax.ShapeDtypeStruct((2, N_PAD, 128), jnp.bfloat16),
                   jax.ShapeDtypeStruct((N_PAD, 16), f32),
                   jax.ShapeDtypeStruct((N_PAD, 16), f32)],
    )(x_p, W1, As, Ad)


def _elu(u):
    return jnp.where(u > 0, u, jnp.exp(jnp.minimum(u, 0.0)) - 1.0)


def _mid_body(pa_ref, pb_ref, den_ref, w2a_ref, w2b_ref, b1a_ref, b1b_ref,
              e1a_ref, e1b_ref, as_w, ad_w, h2_ref, as_ref, ad_ref):
    den = den_ref[0] + den_ref[1]
    da = jnp.maximum(jnp.dot(den, e1a_ref[...], preferred_element_type=f32),
                     1e-30)
    db = jnp.maximum(jnp.dot(den, e1b_ref[...], preferred_element_type=f32),
                     1e-30)
    ua = (pa_ref[0] + pa_ref[1]) / da + b1a_ref[...]
    ub = (pb_ref[0] + pb_ref[1]) / db + b1b_ref[...]
    ha = _elu(ua)
    hb = _elu(ub)
    h2 = (jnp.dot(ha, w2a_ref[...], preferred_element_type=f32)
          + jnp.dot(hb, w2b_ref[...], preferred_element_type=f32))
    h2_ref[...] = h2.astype(jnp.bfloat16)
    as_ref[...] = jnp.dot(h2, as_w[...], preferred_element_type=f32)
    ad_ref[...] = jnp.dot(h2, ad_w[...], preferred_element_type=f32)


def _mid(pa, pb, den, W2a, W2b, b1a, b1b, As2, Ad2):
    full = lambda i: (0, 0)
    row3 = lambda i: (0, i, 0)
    row = lambda i: (i, 0)
    return pl.pallas_call(
        _mid_body,
        grid=(N_PAD // BR,),
        in_specs=[pl.BlockSpec((2, BR, 128), row3),
                  pl.BlockSpec((2, BR, 128), row3),
                  pl.BlockSpec((2, BR, 16), row3),
                  pl.BlockSpec((128, HID), full),
                  pl.BlockSpec((128, HID), full),
                  pl.BlockSpec((1, 128), full),
                  pl.BlockSpec((1, 128), full),
                  pl.BlockSpec((16, 128), full),
                  pl.BlockSpec((16, 128), full),
                  pl.BlockSpec((HID, 16), full),
                  pl.BlockSpec((HID, 16), full)],
        out_specs=[pl.BlockSpec((BR, HID), row),
                   pl.BlockSpec((BR, 16), row),
                   pl.BlockSpec((BR, 16), row)],
        out_shape=[jax.ShapeDtypeStruct((N_PAD, HID), jnp.bfloat16),
                   jax.ShapeDtypeStruct((N_PAD, 16), f32),
                   jax.ShapeDtypeStruct((N_PAD, 16), f32)],
    )(pa, pb, den, W2a, W2b, b1a, b1b, E1A, E1B, As2, Ad2)


def _post_body(p_ref, den_ref, e2_ref, b2_ref, wout_ref, bout_ref, y_ref):
    den = den_ref[0] + den_ref[1]
    d = jnp.maximum(jnp.dot(den, e2_ref[...], preferred_element_type=f32),
                    1e-30)
    u = (p_ref[0] + p_ref[1]) / d + b2_ref[...]
    h = _elu(u)
    y_ref[...] = (jnp.dot(h, wout_ref[...], preferred_element_type=f32)
                  + bout_ref[...])


def _post(p2, den2, b2r, Wout, boutr):
    full = lambda i: (0, 0)
    row3 = lambda i: (0, i, 0)
    row = lambda i: (i, 0)
    return pl.pallas_call(
        _post_body,
        grid=(N_PAD // BR,),
        in_specs=[pl.BlockSpec((2, BR, HID), row3),
                  pl.BlockSpec((2, BR, 16), row3),
                  pl.BlockSpec((16, HID), full),
                  pl.BlockSpec((1, HID), full),
                  pl.BlockSpec((HID, D_OUT), full),
                  pl.BlockSpec((1, D_OUT), full)],
        out_specs=[pl.BlockSpec((BR, D_OUT), row)],
        out_shape=[jax.ShapeDtypeStruct((N_PAD, D_OUT), f32)],
    )(p2, den2, E2, b2r, Wout, boutr)


def _make_edge(d_feat, head_base, compute_den, with_msg, ck, phases=1,
               h_bf16=False):
    nvec = d_feat // 16
    ncht = EPT // ck
    nchp = ncht // phases
    mesh = plsc.VectorSubcoreMesh(core_axis_name="c", subcore_axis_name="s")

    out_type = []
    if with_msg:
        out_type.append(jax.ShapeDtypeStruct((2, N_PAD, d_feat), f32))
    if compute_den:
        out_type.append(jax.ShapeDtypeStruct((2, N_PAD, 16), f32))

    bufw = 2 + (1 if compute_den else 0) + (2 if with_msg else 0) + 2
    scratch = [
        pltpu.VMEM((nchp, ck), i32),
        pltpu.VMEM((nchp, ck), i32),
    ]
    for _b in range(2):
        scratch += [pltpu.VMEM((ck, 16), f32),
                    pltpu.VMEM((ck, 16), f32)]
        if compute_den:
            scratch.append(pltpu.VMEM((ck, 16), f32))
        if with_msg:
            hdt = jnp.bfloat16 if h_bf16 else f32
            scratch += [pltpu.VMEM((ck, d_feat), hdt),
                        pltpu.VMEM((ck, d_feat), f32)]
        scratch += [pltpu.SemaphoreType.DMA, pltpu.SemaphoreType.DMA]
    if with_msg:
        scratch.append(pltpu.VMEM_SHARED((NA, d_feat), f32))
    if compute_den:
        scratch.append(pltpu.VMEM_SHARED((NA, 16), f32))

    def body(*args):
        a = list(args)
        h_hbm = a.pop(0) if with_msg else None
        sa_hbm, da_hbm, src_hbm, dst_hbm = a[:4]
        a = a[4:]
        out_hbm = a.pop(0) if with_msg else None
        den_hbm = a.pop(0) if compute_den else None
        idxs, idxd = a[0], a[1]
        bufs = [a[2 + bufw * b:2 + bufw * (b + 1)] for b in range(2)]
        a = a[2 + 2 * bufw:]
        acc = a.pop(0) if with_msg else None
        denacc = a.pop(0) if compute_den else None

        cid = lax.axis_index("c")
        sid = lax.axis_index("s")
        wid = sid * 2 + cid

        def parts(buf):
            sa_v, da_v = buf[0], buf[1]
            k = 2
            w_v = buf[k] if compute_den else None
            k += 1 if compute_den else 0
            h_v = buf[k] if with_msg else None
            msg_v = buf[k + 1] if with_msg else None
            return sa_v, da_v, w_v, h_v, msg_v, buf[-2], buf[-1]

        _, _, w0, _, msg0, _, _ = parts(bufs[0])

        @pl.loop(0, ck)
        def _(r):
            if with_msg:
                for j in range(nvec):
                    msg0[r, pl.ds(j * 16, 16)] = jnp.zeros((16,), f32)
            if compute_den:
                w0[r, :] = jnp.zeros((16,), f32)

        rb = sid * ROWS_A
        nfull, nrem = ROWS_A // ck, ROWS_A % ck
        for k in range(nfull):
            if with_msg:
                pltpu.sync_copy(msg0, acc.at[pl.ds(rb + k * ck, ck)])
            if compute_den:
                pltpu.sync_copy(w0, denacc.at[pl.ds(rb + k * ck, ck)])
        if nrem:
            if with_msg:
                pltpu.sync_copy(msg0.at[pl.ds(0, nrem)],
                                acc.at[pl.ds(rb + nfull * ck, nrem)])
            if compute_den:
                pltpu.sync_copy(w0.at[pl.ds(0, nrem)],
                                denacc.at[pl.ds(rb + nfull * ck, nrem)])
        plsc.subcore_barrier()

        def start_gathers(c, buf):
            sa_v, da_v, _, h_v, _, gsem, _ = parts(buf)
            pltpu.async_copy(sa_hbm.at[idxs.at[c]], sa_v, gsem)
            pltpu.async_copy(da_hbm.at[idxd.at[c]], da_v, gsem)
            if with_msg:
                pltpu.async_copy(h_hbm.at[idxs.at[c]], h_v, gsem)

        def wait_gathers(c, buf):
            sa_v, da_v, _, h_v, _, gsem, _ = parts(buf)
            pltpu.make_async_copy(sa_hbm.at[idxs.at[c]], sa_v, gsem).wait()
            pltpu.make_async_copy(da_hbm.at[idxd.at[c]], da_v, gsem).wait()
            if with_msg:
                pltpu.make_async_copy(h_hbm.at[idxs.at[c]], h_v,
                                      gsem).wait()

        def compute(c, buf):
            sa_v, da_v, w_v, h_v, msg_v, _, _ = parts(buf)

            @pl.loop(0, ck)
            def _(r):
                e = sa_v[r, :] + da_v[r, :]
                e = jnp.maximum(e, 0.2 * e)
                w = jnp.exp(e)
                if compute_den:
                    w_v[r, :] = w
                if with_msg and h_bf16:
                    for j in range(nvec // 2):
                        lane = head_base + j
                        wj = w.at[jnp.full((16,), lane, i32)].get(
                            mode="promise_in_bounds")
                        pair = plsc.bitcast(h_v[r, pl.ds(j * 32, 32)], i32)
                        lo = plsc.bitcast(pair << 16, f32)
                        hi = plsc.bitcast(
                            pair & jnp.int32(-65536), f32)
                        msg_v[r, pl.ds(j * 32, 16)] = lo * wj
                        msg_v[r, pl.ds(j * 32 + 16, 16)] = hi * wj
                elif with_msg:
                    for j in range(nvec):
                        lane = head_base + j // 2
                        wj = w.at[jnp.full((16,), lane, i32)].get(
                            mode="promise_in_bounds")
                        msg_v[r, pl.ds(j * 16, 16)] = (
                            h_v[r, pl.ds(j * 16, 16)] * wj)

        def start_scatters(c, buf):
            _, _, w_v, _, msg_v, _, ssem = parts(buf)
            if with_msg:
                pltpu.async_copy(msg_v, acc.at[idxd.at[c]], ssem, add=True)
            if compute_den:
                pltpu.async_copy(w_v, denacc.at[idxd.at[c]], ssem,
                                 add=True)

        def wait_scatters(c, buf):
            _, _, w_v, _, msg_v, _, ssem = parts(buf)
            if with_msg:
                pltpu.make_async_copy(msg_v, acc.at[idxd.at[c]],
                                      ssem).wait()
            if compute_den:
                pltpu.make_async_copy(w_v, denacc.at[idxd.at[c]],
                                      ssem).wait()

        for p in range(phases):
            pltpu.sync_copy(src_hbm.at[wid, pl.ds(p * nchp, nchp)], idxs)
            pltpu.sync_copy(dst_hbm.at[wid, pl.ds(p * nchp, nchp)], idxd)
            start_gathers(0, bufs[0])

            @pl.loop(0, nchp, step=2)
            def _(c):
                start_gathers(c + 1, bufs[1])

                @pl.when(c >= 2)
                def _():
                    wait_scatters(c - 2, bufs[0])
                wait_gathers(c, bufs[0])
                compute(c, bufs[0])
                start_scatters(c, bufs[0])

                @pl.when(c + 2 < nchp)
                def _():
                    start_gathers(c + 2, bufs[0])

                @pl.when(c >= 2)
                def _():
                    wait_scatters(c - 1, bufs[1])
                wait_gathers(c + 1, bufs[1])
                compute(c + 1, bufs[1])
                start_scatters(c + 1, bufs[1])

            wait_scatters(nchp - 2, bufs[0])
            wait_scatters(nchp - 1, bufs[1])

        plsc.subcore_barrier()
        if with_msg:
            pltpu.sync_copy(acc.at[pl.ds(rb, ROWS_A)],
                            out_hbm.at[cid, pl.ds(rb, ROWS_A)])
        if compute_den:
            pltpu.sync_copy(denacc.at[pl.ds(rb, ROWS_A)],
                            den_hbm.at[cid, pl.ds(rb, ROWS_A)])

    import dataclasses
    cp = pltpu.CompilerParams(use_tc_tiling_on_sc=False)
    if h_bf16 and "needs_layout_passes" in pltpu.CompilerParams.__dataclass_fields__:
        cp = dataclasses.replace(cp, needs_layout_passes=False)
    return pl.kernel(body, out_type=tuple(out_type), mesh=mesh,
                     scratch_types=tuple(scratch),
                     compiler_params=cp)


_edge1a = _make_edge(128, 0, True, True, C1,
                     h_bf16=True)
_edge1b = _make_edge(128, 4, False, True, C1,
                     h_bf16=True)
_edge2 = _make_edge(HID, 0, True, True, C2,
                    h_bf16=True)


def kernel(x, edge_index, W1, a_src1, a_dst1, b1, W2, a_src2, a_dst2, b2,
           Wout, bout):
    loop = jnp.arange(N, dtype=i32)
    src = jnp.concatenate([edge_index[0].astype(i32), loop])
    dst = jnp.concatenate([edge_index[1].astype(i32), loop])
    npad = E_PAD - NE
    src_f = jnp.concatenate([src, jnp.zeros((npad,), i32)])
    dst_f = jnp.concatenate([dst, jnp.full((npad,), N, i32)])

    def idx_pair(ck):
        return (src_f.reshape(NT, EPT // ck, ck),
                dst_f.reshape(NT, EPT // ck, ck))

    src_1, dst_1 = idx_pair(C1)
    src_2, dst_2 = idx_pair(C2)

    x_p = jnp.pad(x.astype(f32), ((0, N_PAD - N), (0, 0)))
    As1 = _embed_attn(a_src1)
    Ad1 = _embed_attn(a_dst1)
    As2 = _embed_attn(a_src2)
    Ad2 = _embed_attn(a_dst2)

    h12, asrc1, adst1 = _pre1(x_p, W1.astype(f32), As1, Ad1)

    pa, den1 = _edge1a(h12[0], asrc1, adst1, src_1, dst_1)
    (pb,) = _edge1b(h12[1], asrc1, adst1, src_1, dst_1)

    h2, asrc2, adst2 = _mid(pa, pb, den1,
                            W2[:128][PERM128].astype(f32),
                            W2[128:][PERM128].astype(f32),
                            b1[:128][PERM128].reshape(1, 128).astype(f32),
                            b1[128:][PERM128].reshape(1, 128).astype(f32),
                            As2, Ad2)

    p2, den2 = _edge2(h2, asrc2, adst2, src_2, dst_2)

    (y,) = _post(p2, den2, b2[_pb32].reshape(1, HID).astype(f32),
                 Wout[_pb32].astype(f32),
                 bout.reshape(1, D_OUT).astype(f32))
    return y[:N]
